# baseline (speedup 1.0000x reference)
import jax
import jax.numpy as jnp
from jax import lax
from jax.experimental import pallas as pl
from jax.experimental.pallas import tpu as pltpu

COARSE = 0.3
MEDIUM = 0.3
N16 = 64 * 32 * 32
N8 = 64 * 64 * 64
K_COARSE = round(N16 * COARSE)
K_MED = round(4 * N16 * COARSE + N8 * MEDIUM)

def _ordered_bits(x):
    b = lax.bitcast_convert_type(x, jnp.int32)
    return b ^ (lax.shift_right_arithmetic(b, 31) & 2147483647)


def _bits_to_f32(o):
    b = o ^ (lax.shift_right_arithmetic(o, 31) & 2147483647)
    return lax.bitcast_convert_type(b, jnp.float32)


def _kth_smallest(o_ref, k):
    cnt_neg = jnp.sum((o_ref[...] < 0).astype(jnp.int32))
    base = jnp.where(k <= cnt_neg, jnp.int32(-2147483648), jnp.int32(0))

    def body(i, prefix):
        b = jnp.int32(30) - i
        low = lax.shift_left(jnp.int32(1), b) - 1
        test = base + prefix + low
        cnt = jnp.sum((o_ref[...] <= test).astype(jnp.int32))
        bit = jnp.where(cnt >= k, jnp.int32(0), lax.shift_left(jnp.int32(1), b))
        return prefix + bit

    prefix = lax.fori_loop(0, 31, body, jnp.int32(0))
    return base + prefix


def _thr_body(p16_ref, p8_ref, p16u2_ref, thr_ref, o16_ref, o8_ref):
    o16_ref[...] = _ordered_bits(p16_ref[...])
    t16b = _kth_smallest(o16_ref, K_COARSE)
    thr16 = _bits_to_f32(t16b)

    gc_up = (p16u2_ref[...] < thr16).astype(jnp.float32)
    p8m = p8_ref[...] * (1.0 - gc_up)
    o8_ref[...] = _ordered_bits(p8m)
    t8b = _kth_smallest(o8_ref, K_MED)
    thr8 = _bits_to_f32(t8b)

    row = lax.broadcasted_iota(jnp.int32, (8, 128), 0)
    col = lax.broadcasted_iota(jnp.int32, (8, 128), 1)
    out = jnp.where((row == 0) & (col == 0), thr16,
                    jnp.where((row == 0) & (col == 1), thr8, 0.0))
    thr_ref[...] = out


def _up(x, s):
    return jnp.repeat(jnp.repeat(x, s, axis=1), s, axis=2)


def _gate_body(thr_ref, p16_ref, p8_ref, m0_ref, m1_ref, m2_ref, gate_ref):
    t16 = thr_ref[0, 0]
    t8 = thr_ref[0, 1]
    p16 = p16_ref[...]
    p8 = p8_ref[...]

    gc = p16 < t16
    gcf = gc.astype(jnp.float32)
    m0_ref[:, 0] = gc.astype(jnp.int32)

    u2 = _up(gcf, 2)
    gm = (p8 < t8) & (u2 == 0.0)
    gmf = gm.astype(jnp.float32)
    m1_ref[:, 0] = gm.astype(jnp.int32)

    cf = _up(gcf, 4)
    mf = _up(gmf, 2)
    ff = 1.0 - cf - mf
    m2_ref[:, 0] = (ff != 0.0).astype(jnp.int32)
    gate_ref[:, 0, :, 0:128] = cf
    gate_ref[:, 0, :, 128:256] = mf
    gate_ref[:, 0, :, 256:384] = ff


def _make_thr_call(interpret=False):
    return pl.pallas_call(
        _thr_body,
        out_shape=jax.ShapeDtypeStruct((8, 128), jnp.float32),
        scratch_shapes=[
            pltpu.VMEM((512, 128), jnp.int32),
            pltpu.VMEM((2048, 128), jnp.int32),
        ],
        interpret=interpret,
    )


def _make_gate_call(interpret=False):
    bb = 8
    return pl.pallas_call(
        _gate_body,
        grid=(64 // bb,),
        in_specs=[
            pl.BlockSpec((8, 128), lambda b: (0, 0)),
            pl.BlockSpec((bb, 32, 32), lambda b: (b, 0, 0)),
            pl.BlockSpec((bb, 64, 64), lambda b: (b, 0, 0)),
        ],
        out_specs=[
            pl.BlockSpec((bb, 1, 32, 32), lambda b: (b, 0, 0, 0)),
            pl.BlockSpec((bb, 1, 64, 64), lambda b: (b, 0, 0, 0)),
            pl.BlockSpec((bb, 1, 128, 128), lambda b: (b, 0, 0, 0)),
            pl.BlockSpec((bb, 1, 128, 384), lambda b: (b, 0, 0, 0)),
        ],
        out_shape=[
            jax.ShapeDtypeStruct((64, 1, 32, 32), jnp.int32),
            jax.ShapeDtypeStruct((64, 1, 64, 64), jnp.int32),
            jax.ShapeDtypeStruct((64, 1, 128, 128), jnp.int32),
            jax.ShapeDtypeStruct((64, 1, 128, 384), jnp.float32),
        ],
        interpret=interpret,
    )


def _kernel_impl(x_entropy_p16, x_entropy_p8, interpret=False):
    p16f = x_entropy_p16.reshape(512, 128)
    p8f = x_entropy_p8.reshape(2048, 128)
    p16u2 = jnp.repeat(jnp.repeat(x_entropy_p16, 2, axis=1), 2, axis=2)
    p16u2f = p16u2.reshape(2048, 128)

    thr = _make_thr_call(interpret)(p16f, p8f, p16u2f)
    m0, m1, m2, gate = _make_gate_call(interpret)(thr, x_entropy_p16, x_entropy_p8)
    return m0, m1, m2, gate


@jax.jit
def kernel(x_entropy_p16, x_entropy_p8):
    return _kernel_impl(x_entropy_p16, x_entropy_p8)


# speedup vs baseline: 2.0465x; 2.0465x over previous
import jax
import jax.numpy as jnp
from jax import lax
from jax.experimental import pallas as pl
from jax.experimental.pallas import tpu as pltpu

COARSE = 0.3
MEDIUM = 0.3
N16 = 64 * 32 * 32
N8 = 64 * 64 * 64
K_COARSE = round(N16 * COARSE)
K_MED = round(4 * N16 * COARSE + N8 * MEDIUM)

def _ordered_bits(x):
    b = lax.bitcast_convert_type(x, jnp.int32)
    return b ^ (lax.shift_right_arithmetic(b, 31) & 2147483647)


def _bits_to_f32(o):
    b = o ^ (lax.shift_right_arithmetic(o, 31) & 2147483647)
    return lax.bitcast_convert_type(b, jnp.float32)


def _kth_smallest(o_ref, k):
    cnt_neg = jnp.sum((o_ref[...] < 0).astype(jnp.int32))
    base = jnp.where(k <= cnt_neg, jnp.int32(-2147483648), jnp.int32(0))

    def body(i, prefix):
        b = jnp.int32(30) - i
        low = lax.shift_left(jnp.int32(1), b) - 1
        test = base + prefix + low
        cnt = jnp.sum((o_ref[...] <= test).astype(jnp.int32))
        bit = jnp.where(cnt >= k, jnp.int32(0), lax.shift_left(jnp.int32(1), b))
        return prefix + bit

    prefix = lax.fori_loop(0, 31, body, jnp.int32(0))
    return base + prefix


def _thr_body(p16_ref, p8_ref, p16u2_ref, thr_ref, o16_ref, o8_ref):
    o16_ref[...] = _ordered_bits(p16_ref[...])
    t16b = _kth_smallest(o16_ref, K_COARSE)
    thr16 = _bits_to_f32(t16b)

    gc_up = (p16u2_ref[...] < thr16).astype(jnp.float32)
    p8m = p8_ref[...] * (1.0 - gc_up)
    o8_ref[...] = _ordered_bits(p8m)
    t8b = _kth_smallest(o8_ref, K_MED)
    thr8 = _bits_to_f32(t8b)

    row = lax.broadcasted_iota(jnp.int32, (8, 128), 0)
    col = lax.broadcasted_iota(jnp.int32, (8, 128), 1)
    out = jnp.where((row == 0) & (col == 0), thr16,
                    jnp.where((row == 0) & (col == 1), thr8, 0.0))
    thr_ref[...] = out


def _gate_body(thr_ref, p16_ref, p8_ref, p16u2_ref, p16u4_ref, p8u2_ref,
               m0_ref, m1_ref, m2_ref, gate_ref):
    t16 = thr_ref[0, 0]
    t8 = thr_ref[0, 1]

    m0_ref[:, 0] = (p16_ref[...] < t16).astype(jnp.int32)

    u2 = p16u2_ref[...] < t16
    gm = (p8_ref[...] < t8) & ~u2
    m1_ref[:, 0] = gm.astype(jnp.int32)

    cf = (p16u4_ref[...] < t16).astype(jnp.float32)
    mf = ((p8u2_ref[...] < t8) & ~(p16u4_ref[...] < t16)).astype(jnp.float32)
    ff = 1.0 - cf - mf
    m2_ref[:, 0] = (ff != 0.0).astype(jnp.int32)
    gate_ref[:, 0, :, 0:128] = cf
    gate_ref[:, 0, :, 128:256] = mf
    gate_ref[:, 0, :, 256:384] = ff


def _make_thr_call(interpret=False):
    return pl.pallas_call(
        _thr_body,
        out_shape=jax.ShapeDtypeStruct((8, 128), jnp.float32),
        scratch_shapes=[
            pltpu.VMEM((512, 128), jnp.int32),
            pltpu.VMEM((2048, 128), jnp.int32),
        ],
        interpret=interpret,
    )


def _make_gate_call(interpret=False):
    bb = 8
    return pl.pallas_call(
        _gate_body,
        grid=(64 // bb,),
        in_specs=[
            pl.BlockSpec((8, 128), lambda b: (0, 0)),
            pl.BlockSpec((bb, 32, 32), lambda b: (b, 0, 0)),
            pl.BlockSpec((bb, 64, 64), lambda b: (b, 0, 0)),
            pl.BlockSpec((bb, 64, 64), lambda b: (b, 0, 0)),
            pl.BlockSpec((bb, 128, 128), lambda b: (b, 0, 0)),
            pl.BlockSpec((bb, 128, 128), lambda b: (b, 0, 0)),
        ],
        out_specs=[
            pl.BlockSpec((bb, 1, 32, 32), lambda b: (b, 0, 0, 0)),
            pl.BlockSpec((bb, 1, 64, 64), lambda b: (b, 0, 0, 0)),
            pl.BlockSpec((bb, 1, 128, 128), lambda b: (b, 0, 0, 0)),
            pl.BlockSpec((bb, 1, 128, 384), lambda b: (b, 0, 0, 0)),
        ],
        out_shape=[
            jax.ShapeDtypeStruct((64, 1, 32, 32), jnp.int32),
            jax.ShapeDtypeStruct((64, 1, 64, 64), jnp.int32),
            jax.ShapeDtypeStruct((64, 1, 128, 128), jnp.int32),
            jax.ShapeDtypeStruct((64, 1, 128, 384), jnp.float32),
        ],
        interpret=interpret,
    )


def _kernel_impl(x_entropy_p16, x_entropy_p8, interpret=False):
    p16f = x_entropy_p16.reshape(512, 128)
    p8f = x_entropy_p8.reshape(2048, 128)
    p16u2 = jnp.repeat(jnp.repeat(x_entropy_p16, 2, axis=1), 2, axis=2)
    p16u2f = p16u2.reshape(2048, 128)
    p16u4 = jnp.repeat(jnp.repeat(x_entropy_p16, 4, axis=1), 4, axis=2)
    p8u2 = jnp.repeat(jnp.repeat(x_entropy_p8, 2, axis=1), 2, axis=2)

    thr = _make_thr_call(interpret)(p16f, p8f, p16u2f)
    m0, m1, m2, gate = _make_gate_call(interpret)(
        thr, x_entropy_p16, x_entropy_p8, p16u2, p16u4, p8u2)
    return m0, m1, m2, gate


@jax.jit
def kernel(x_entropy_p16, x_entropy_p8):
    return _kernel_impl(x_entropy_p16, x_entropy_p8)


# speedup vs baseline: 3.8468x; 1.8797x over previous
import jax
import jax.numpy as jnp
from jax import lax
from jax.experimental import pallas as pl
from jax.experimental.pallas import tpu as pltpu

COARSE = 0.3
MEDIUM = 0.3
N16 = 64 * 32 * 32
N8 = 64 * 64 * 64
K_COARSE = round(N16 * COARSE)
K_MED = round(4 * N16 * COARSE + N8 * MEDIUM)

BB = 8


def _ordered_bits(x):
    b = lax.bitcast_convert_type(x, jnp.int32)
    return b ^ (lax.shift_right_arithmetic(b, 31) & 2147483647)


def _bits_to_f32(o):
    b = o ^ (lax.shift_right_arithmetic(o, 31) & 2147483647)
    return lax.bitcast_convert_type(b, jnp.float32)


def _kth_smallest(o_ref, k):
    cnt_neg = jnp.sum((o_ref[...] < 0).astype(jnp.int32))
    base = jnp.where(k <= cnt_neg, jnp.int32(-2147483648), jnp.int32(0))

    def body(i, prefix):
        b = jnp.int32(30) - i
        low = lax.shift_left(jnp.int32(1), b) - 1
        test = base + prefix + low
        cnt = jnp.sum((o_ref[...] <= test).astype(jnp.int32))
        bit = jnp.where(cnt >= k, jnp.int32(0), lax.shift_left(jnp.int32(1), b))
        return prefix + bit

    prefix = lax.fori_loop(0, 31, body, jnp.int32(0))
    return base + prefix


def _body(p16_ref, p8_ref, p16u2_ref, p16u4_ref, p8u2_ref,
          m0_ref, m1_ref, m2_ref, gate_ref, o16_ref, o8_ref, thr_ref):
    i = pl.program_id(0)

    @pl.when(i == 0)
    def _thresholds():
        o16_ref[...] = _ordered_bits(p16_ref[...])
        thr16 = _bits_to_f32(_kth_smallest(o16_ref, K_COARSE))
        thr_ref[0] = thr16
        gc_up = (p16u2_ref[...] < thr16).astype(jnp.float32)
        o8_ref[...] = _ordered_bits(p8_ref[...] * (1.0 - gc_up))
        thr_ref[1] = _bits_to_f32(_kth_smallest(o8_ref, K_MED))

    @pl.when(i > 0)
    def _gate():
        t16 = thr_ref[0]
        t8 = thr_ref[1]
        b0 = (i - 1) * BB
        m0_ref[:, 0] = (p16_ref[pl.ds(b0, BB)] < t16).astype(jnp.int32)

        u2 = p16u2_ref[pl.ds(b0, BB)] < t16
        gm = (p8_ref[pl.ds(b0, BB)] < t8) & ~u2
        m1_ref[:, 0] = gm.astype(jnp.int32)

        cfb = p16u4_ref[...] < t16
        cf = cfb.astype(jnp.float32)
        mf = ((p8u2_ref[...] < t8) & ~cfb).astype(jnp.float32)
        ff = 1.0 - cf - mf
        m2_ref[:, 0] = (ff != 0.0).astype(jnp.int32)
        gate_ref[:, 0, :, 0:128] = cf
        gate_ref[:, 0, :, 128:256] = mf
        gate_ref[:, 0, :, 256:384] = ff


def _make_call(interpret=False):
    def gidx(i):
        return (jnp.maximum(i - 1, 0), 0, 0)

    def gidx4(i):
        return (jnp.maximum(i - 1, 0), 0, 0, 0)

    return pl.pallas_call(
        _body,
        grid=(1 + 64 // BB,),
        in_specs=[
            pl.BlockSpec((64, 32, 32), lambda i: (0, 0, 0)),
            pl.BlockSpec((64, 64, 64), lambda i: (0, 0, 0)),
            pl.BlockSpec((64, 64, 64), lambda i: (0, 0, 0)),
            pl.BlockSpec((BB, 128, 128), gidx),
            pl.BlockSpec((BB, 128, 128), gidx),
        ],
        out_specs=[
            pl.BlockSpec((BB, 1, 32, 32), gidx4),
            pl.BlockSpec((BB, 1, 64, 64), gidx4),
            pl.BlockSpec((BB, 1, 128, 128), gidx4),
            pl.BlockSpec((BB, 1, 128, 384), gidx4),
        ],
        out_shape=[
            jax.ShapeDtypeStruct((64, 1, 32, 32), jnp.int32),
            jax.ShapeDtypeStruct((64, 1, 64, 64), jnp.int32),
            jax.ShapeDtypeStruct((64, 1, 128, 128), jnp.int32),
            jax.ShapeDtypeStruct((64, 1, 128, 384), jnp.float32),
        ],
        scratch_shapes=[
            pltpu.VMEM((64, 32, 32), jnp.int32),
            pltpu.VMEM((64, 64, 64), jnp.int32),
            pltpu.SMEM((2,), jnp.float32),
        ],
        interpret=interpret,
    )


def _kernel_impl(x_entropy_p16, x_entropy_p8, interpret=False):
    p16u2 = jnp.repeat(jnp.repeat(x_entropy_p16, 2, axis=1), 2, axis=2)
    p16u4 = jnp.repeat(jnp.repeat(x_entropy_p16, 4, axis=1), 4, axis=2)
    p8u2 = jnp.repeat(jnp.repeat(x_entropy_p8, 2, axis=1), 2, axis=2)
    return _make_call(interpret)(x_entropy_p16, x_entropy_p8, p16u2, p16u4, p8u2)


@jax.jit
def kernel(x_entropy_p16, x_entropy_p8):
    return _kernel_impl(x_entropy_p16, x_entropy_p8)
